# baseline (speedup 1.0000x reference)
import functools

import numpy as np
import jax
import jax.numpy as jnp
from jax import lax
from jax.experimental import pallas as pl
from jax.experimental.pallas import tpu as pltpu
from jax.experimental.pallas import tpu_sc as plsc

NSEG = 1024
N = 320000
D = 128
NC = 2
NS = 16
NW = NC * NS
ROWS_PER_W = N // NW
CHUNK = 80
NGRP = CHUNK // 16
NCHUNK = ROWS_PER_W // CHUNK
SEG_PER_TILE = NSEG // NS
TRASH = NSEG
ACC_ROWS = NSEG + 16
CSTAGE = CHUNK + 16


def _sc_body(prev_hbm, embs_hbm, idx_hbm, out_hbm,
             pbuf, ebuf, ibuf, cbuf_p, cbuf_e, fidx, idxb, zbuf,
             acc_p, acc_e, sem_d0, sem_d1):
    core = lax.axis_index("c")
    sid = lax.axis_index("s")
    wid = core * NS + sid
    row_base = wid * ROWS_PER_W
    sem_d = (sem_d0, sem_d1)

    iota = jnp.arange(16, dtype=jnp.int32)
    zeros16 = jnp.zeros((16,), jnp.float32)
    trash16 = jnp.full((16,), TRASH, jnp.int32)

    def zero_row(r, _):
        for c in range(D // 16):
            zbuf[r, pl.ds(c * 16, 16)] = zeros16
        return _

    lax.fori_loop(0, SEG_PER_TILE, zero_row, None)
    seg_lo = sid * SEG_PER_TILE
    pltpu.sync_copy(zbuf, acc_p.at[pl.ds(seg_lo, SEG_PER_TILE)])
    pltpu.sync_copy(zbuf, acc_e.at[pl.ds(seg_lo, SEG_PER_TILE)])
    plsc.subcore_barrier()

    def issue(b, i):
        base = row_base + i * CHUNK
        pltpu.async_copy(prev_hbm.at[pl.ds(base, CHUNK)], pbuf.at[b], sem_d[b])
        pltpu.async_copy(embs_hbm.at[pl.ds(base, CHUNK)], ebuf.at[b], sem_d[b])
        pltpu.async_copy(idx_hbm.at[pl.ds(base, CHUNK)], ibuf.at[b], sem_d[b])

    def wait_dma(b):
        pltpu.make_async_copy(prev_hbm.at[pl.ds(0, CHUNK)], pbuf.at[b], sem_d[b]).wait()
        pltpu.make_async_copy(embs_hbm.at[pl.ds(0, CHUNK)], ebuf.at[b], sem_d[b]).wait()
        pltpu.make_async_copy(idx_hbm.at[pl.ds(0, CHUNK)], ibuf.at[b], sem_d[b]).wait()

    issue(0, 0)
    issue(1, 1)

    def process_chunk(b, i, carry):
        pacc, eacc, last_idx = carry
        wait_dma(b)

        for c in range(D // 16):
            cbuf_p[0, pl.ds(c * 16, 16)] = pacc[c]
            cbuf_e[0, pl.ds(c * 16, 16)] = eacc[c]
        for t in range(CSTAGE // 16):
            fidx[pl.ds(16 * t, 16)] = trash16

        def group_body(g, gcarry):
            pacc, eacc, last_idx, pos = gcarry
            iv = ibuf[b, pl.ds(16 * g, 16)]
            prev = plsc.load_gather(
                ibuf.at[b], [jnp.maximum(16 * g + iota - 1, 0)])
            prev = jnp.where((iota == 0) & (g == 0), last_idx, prev)
            prev = jnp.where(prev < 0, TRASH, prev)
            bmask = iv != prev
            bmask_i = bmask.astype(jnp.int32)
            bits = jnp.sum(bmask_i << iota)
            plsc.store_compressed(fidx.at[pl.ds(pos, 16)], prev, mask=bmask)
            last_idx = jnp.sum(jnp.where(iota == 15, iv, 0))

            for j in range(16):
                r = 16 * g + j
                bit = (bits >> j) & 1
                pos = pos + bit
                m = (jnp.zeros((16,), jnp.int32) + bit) > 0
                new_p = []
                new_e = []
                for c in range(D // 16):
                    row = pbuf[b, r, pl.ds(c * 16, 16)]
                    a = jnp.where(m, row, pacc[c] + row)
                    cbuf_p[pos, pl.ds(c * 16, 16)] = a
                    new_p.append(a)
                for c in range(D // 16):
                    row = ebuf[b, r, pl.ds(c * 16, 16)]
                    a = jnp.where(m, row, eacc[c] + row)
                    cbuf_e[pos, pl.ds(c * 16, 16)] = a
                    new_e.append(a)
                pacc, eacc = tuple(new_p), tuple(new_e)
            return pacc, eacc, last_idx, pos

        pacc, eacc, last_idx, pos = lax.fori_loop(
            0, NGRP, group_body, (pacc, eacc, last_idx, jnp.int32(0)))

        @pl.when(i + 2 < NCHUNK)
        def _():
            issue(b, i + 2)

        n = pos

        @pl.when(n > 0)
        def _():
            for t in range(NGRP):
                idxb[t, pl.ds(0, 16)] = fidx[pl.ds(16 * t, 16)]
            for t in range(NGRP):
                @pl.when(n > 16 * t)
                def _(t=t):
                    pltpu.sync_copy(cbuf_p.at[pl.ds(16 * t, 16)],
                                    acc_p.at[idxb.at[t]], add=True)
                    pltpu.sync_copy(cbuf_e.at[pl.ds(16 * t, 16)],
                                    acc_e.at[idxb.at[t]], add=True)

        return pacc, eacc, last_idx

    def pair_body(g, carry):
        carry = process_chunk(0, 2 * g, carry)
        carry = process_chunk(1, 2 * g + 1, carry)
        return carry

    acc0 = tuple(zeros16 for _ in range(D // 16))
    carry = lax.fori_loop(0, NCHUNK // 2, pair_body,
                          (acc0, acc0, jnp.int32(-1)))
    pacc, eacc, last_idx = process_chunk(0, jnp.int32(NCHUNK - 1), carry)

    for c in range(D // 16):
        cbuf_p[0, pl.ds(c * 16, 16)] = pacc[c]
        cbuf_e[0, pl.ds(c * 16, 16)] = eacc[c]
    idxb[0, pl.ds(0, 16)] = jnp.where(iota == 0, last_idx, TRASH)
    pltpu.sync_copy(cbuf_p.at[pl.ds(0, 16)], acc_p.at[idxb.at[0]], add=True)
    pltpu.sync_copy(cbuf_e.at[pl.ds(0, 16)], acc_e.at[idxb.at[0]], add=True)

    plsc.subcore_barrier()
    pltpu.sync_copy(acc_p.at[pl.ds(seg_lo, SEG_PER_TILE)],
                    out_hbm.at[core, 0, pl.ds(seg_lo, SEG_PER_TILE)])
    pltpu.sync_copy(acc_e.at[pl.ds(seg_lo, SEG_PER_TILE)],
                    out_hbm.at[core, 1, pl.ds(seg_lo, SEG_PER_TILE)])


_sc_segment_sums = functools.partial(
    pl.kernel,
    out_type=jax.ShapeDtypeStruct((NC, 2, NSEG, D), jnp.float32),
    mesh=plsc.VectorSubcoreMesh(core_axis_name="c", subcore_axis_name="s",
                                num_cores=NC, num_subcores=NS),
    compiler_params=pltpu.CompilerParams(needs_layout_passes=False),
    scratch_types=[
        pltpu.VMEM((2, CHUNK, D), jnp.float32),
        pltpu.VMEM((2, CHUNK, D), jnp.float32),
        pltpu.VMEM((2, CHUNK), jnp.int32),
        pltpu.VMEM((CSTAGE, D), jnp.float32),
        pltpu.VMEM((CSTAGE, D), jnp.float32),
        pltpu.VMEM((CSTAGE,), jnp.int32),
        pltpu.VMEM((NGRP, 16), jnp.int32),
        pltpu.VMEM((SEG_PER_TILE, D), jnp.float32),
        pltpu.VMEM_SHARED((ACC_ROWS, D), jnp.float32),
        pltpu.VMEM_SHARED((ACC_ROWS, D), jnp.float32),
        pltpu.SemaphoreType.DMA,
        pltpu.SemaphoreType.DMA,
    ],
)(_sc_body)


def _tc_body(part_ref, w_ref, b_ref, out_ref):
    p = part_ref[...]
    agg1 = p[0, 0] + p[1, 0]
    agg2 = p[0, 1] + p[1, 1]
    w = w_ref[...]
    out_ref[...] = (
        jnp.dot(agg1, w[:, :D].T, preferred_element_type=jnp.float32)
        + jnp.dot(agg2, w[:, D:].T, preferred_element_type=jnp.float32)
        + b_ref[...]
    )


def kernel(embs, prev_h, batch_idx, W, b):
    partials = _sc_segment_sums(prev_h, embs, batch_idx)
    out = pl.pallas_call(
        _tc_body,
        out_shape=jax.ShapeDtypeStruct((NSEG, D), jnp.float32),
    )(partials, W, b.reshape(1, D))
    return out


# speedup vs baseline: 4.2935x; 4.2935x over previous
import functools

import numpy as np
import jax
import jax.numpy as jnp
from jax import lax
from jax.experimental import pallas as pl
from jax.experimental.pallas import tpu as pltpu
from jax.experimental.pallas import tpu_sc as plsc

NSEG = 1024
N = 320000
D = 128
NC = 2
NS = 16
NW = NC * NS
ROWS_PER_W = N // NW
CHUNK = 80
NGRP = CHUNK // 16
NCHUNK = ROWS_PER_W // CHUNK
SEG_PER_TILE = NSEG // NS
TRASH = NSEG
ACC_ROWS = NSEG + 16
CSTAGE = CHUNK + 16


def _sc_body(prev_hbm, embs_hbm, idx_hbm, out_hbm,
             pbuf, ebuf, ibuf, cbuf_p, cbuf_e, fidx, idxb, zbuf,
             acc_p, acc_e, sem_d0, sem_d1):
    core = lax.axis_index("c")
    sid = lax.axis_index("s")
    wid = core * NS + sid
    row_base = wid * ROWS_PER_W
    sem_d = (sem_d0, sem_d1)

    iota = jnp.arange(16, dtype=jnp.int32)
    zeros16 = jnp.zeros((16,), jnp.float32)
    trash16 = jnp.full((16,), TRASH, jnp.int32)

    def zero_row(r, _):
        for c in range(D // 16):
            zbuf[r, pl.ds(c * 16, 16)] = zeros16
        return _

    lax.fori_loop(0, SEG_PER_TILE, zero_row, None)
    seg_lo = sid * SEG_PER_TILE
    pltpu.sync_copy(zbuf, acc_p.at[pl.ds(seg_lo, SEG_PER_TILE)])
    pltpu.sync_copy(zbuf, acc_e.at[pl.ds(seg_lo, SEG_PER_TILE)])
    plsc.subcore_barrier()

    def issue(b, i):
        base = row_base + i * CHUNK
        pltpu.async_copy(prev_hbm.at[pl.ds(base, CHUNK)], pbuf.at[b], sem_d[b])
        pltpu.async_copy(embs_hbm.at[pl.ds(base, CHUNK)], ebuf.at[b], sem_d[b])
        pltpu.async_copy(idx_hbm.at[pl.ds(base, CHUNK)], ibuf.at[b], sem_d[b])

    def wait_dma(b):
        pltpu.make_async_copy(prev_hbm.at[pl.ds(0, CHUNK)], pbuf.at[b], sem_d[b]).wait()
        pltpu.make_async_copy(embs_hbm.at[pl.ds(0, CHUNK)], ebuf.at[b], sem_d[b]).wait()
        pltpu.make_async_copy(idx_hbm.at[pl.ds(0, CHUNK)], ibuf.at[b], sem_d[b]).wait()

    issue(0, 0)
    issue(1, 1)

    def process_chunk(b, i, carry):
        pacc, eacc, last_idx = carry
        wait_dma(b)

        for t in range(CSTAGE // 16):
            fidx[pl.ds(16 * t, 16)] = trash16

        def group_body(g, gcarry):
            pacc, eacc, last_idx, pos = gcarry
            iv = ibuf[b, pl.ds(16 * g, 16)]
            prev = plsc.load_gather(
                ibuf.at[b], [jnp.maximum(16 * g + iota - 1, 0)])
            prev = jnp.where((iota == 0) & (g == 0), last_idx, prev)
            prev = jnp.where(prev < 0, TRASH, prev)
            bmask = iv != prev
            bmask_i = bmask.astype(jnp.int32)
            bits = jnp.sum(bmask_i << iota)
            plsc.store_compressed(fidx.at[pl.ds(pos, 16)], prev, mask=bmask)
            last_idx = jnp.sum(jnp.where(iota == 15, iv, 0))

            for j in range(16):
                r = 16 * g + j
                bit = (bits >> j) & 1

                @pl.when(bit == 1)
                def _(pacc=pacc, eacc=eacc, pos=pos):
                    for c in range(D // 16):
                        cbuf_p[pos, pl.ds(c * 16, 16)] = pacc[c]
                    for c in range(D // 16):
                        cbuf_e[pos, pl.ds(c * 16, 16)] = eacc[c]

                pos = pos + bit
                m = (jnp.zeros((16,), jnp.int32) + bit) > 0
                new_p = []
                new_e = []
                for c in range(D // 16):
                    row = pbuf[b, r, pl.ds(c * 16, 16)]
                    new_p.append(jnp.where(m, row, pacc[c] + row))
                for c in range(D // 16):
                    row = ebuf[b, r, pl.ds(c * 16, 16)]
                    new_e.append(jnp.where(m, row, eacc[c] + row))
                pacc, eacc = tuple(new_p), tuple(new_e)
            return pacc, eacc, last_idx, pos

        pacc, eacc, last_idx, pos = lax.fori_loop(
            0, NGRP, group_body, (pacc, eacc, last_idx, jnp.int32(0)))

        @pl.when(i + 2 < NCHUNK)
        def _():
            issue(b, i + 2)

        n = pos

        @pl.when(n > 0)
        def _():
            for t in range(NGRP):
                idxb[t, pl.ds(0, 16)] = fidx[pl.ds(16 * t, 16)]
            for t in range(NGRP):
                @pl.when(n > 16 * t)
                def _(t=t):
                    pltpu.sync_copy(cbuf_p.at[pl.ds(16 * t, 16)],
                                    acc_p.at[idxb.at[t]], add=True)
                    pltpu.sync_copy(cbuf_e.at[pl.ds(16 * t, 16)],
                                    acc_e.at[idxb.at[t]], add=True)

        return pacc, eacc, last_idx

    def pair_body(g, carry):
        carry = process_chunk(0, 2 * g, carry)
        carry = process_chunk(1, 2 * g + 1, carry)
        return carry

    acc0 = tuple(zeros16 for _ in range(D // 16))
    carry = lax.fori_loop(0, NCHUNK // 2, pair_body,
                          (acc0, acc0, jnp.int32(-1)))
    pacc, eacc, last_idx = process_chunk(0, jnp.int32(NCHUNK - 1), carry)

    for c in range(D // 16):
        cbuf_p[0, pl.ds(c * 16, 16)] = pacc[c]
        cbuf_e[0, pl.ds(c * 16, 16)] = eacc[c]
    idxb[0, pl.ds(0, 16)] = jnp.where(iota == 0, last_idx, TRASH)
    pltpu.sync_copy(cbuf_p.at[pl.ds(0, 16)], acc_p.at[idxb.at[0]], add=True)
    pltpu.sync_copy(cbuf_e.at[pl.ds(0, 16)], acc_e.at[idxb.at[0]], add=True)

    plsc.subcore_barrier()
    pltpu.sync_copy(acc_p.at[pl.ds(seg_lo, SEG_PER_TILE)],
                    out_hbm.at[core, 0, pl.ds(seg_lo, SEG_PER_TILE)])
    pltpu.sync_copy(acc_e.at[pl.ds(seg_lo, SEG_PER_TILE)],
                    out_hbm.at[core, 1, pl.ds(seg_lo, SEG_PER_TILE)])


_sc_segment_sums = functools.partial(
    pl.kernel,
    out_type=jax.ShapeDtypeStruct((NC, 2, NSEG, D), jnp.float32),
    mesh=plsc.VectorSubcoreMesh(core_axis_name="c", subcore_axis_name="s",
                                num_cores=NC, num_subcores=NS),
    compiler_params=pltpu.CompilerParams(needs_layout_passes=False),
    scratch_types=[
        pltpu.VMEM((2, CHUNK, D), jnp.float32),
        pltpu.VMEM((2, CHUNK, D), jnp.float32),
        pltpu.VMEM((2, CHUNK), jnp.int32),
        pltpu.VMEM((CSTAGE, D), jnp.float32),
        pltpu.VMEM((CSTAGE, D), jnp.float32),
        pltpu.VMEM((CSTAGE,), jnp.int32),
        pltpu.VMEM((NGRP, 16), jnp.int32),
        pltpu.VMEM((SEG_PER_TILE, D), jnp.float32),
        pltpu.VMEM_SHARED((ACC_ROWS, D), jnp.float32),
        pltpu.VMEM_SHARED((ACC_ROWS, D), jnp.float32),
        pltpu.SemaphoreType.DMA,
        pltpu.SemaphoreType.DMA,
    ],
)(_sc_body)


def _tc_body(part_ref, w_ref, b_ref, out_ref):
    p = part_ref[...]
    agg1 = p[0, 0] + p[1, 0]
    agg2 = p[0, 1] + p[1, 1]
    w = w_ref[...]
    out_ref[...] = (
        jnp.dot(agg1, w[:, :D].T, preferred_element_type=jnp.float32)
        + jnp.dot(agg2, w[:, D:].T, preferred_element_type=jnp.float32)
        + b_ref[...]
    )


def kernel(embs, prev_h, batch_idx, W, b):
    partials = _sc_segment_sums(prev_h, embs, batch_idx)
    out = pl.pallas_call(
        _tc_body,
        out_shape=jax.ShapeDtypeStruct((NSEG, D), jnp.float32),
    )(partials, W, b.reshape(1, D))
    return out
